# baseline (speedup 1.0000x reference)
import functools

import jax
import jax.numpy as jnp
from jax import lax
from jax.experimental import pallas as pl
from jax.experimental.pallas import tpu as pltpu

_NEMB = 4
_NFEATS = 15
_NHIDDEN = 128
_NEWDIM = 3 * _NEMB + _NEMB + _NEMB + _NFEATS
_NTAB = 5
_EPS = 1e-5


def _board_kernel(intsT_ref, featsT_ref, gmap_ref, waug_ref, out_ref):
    R = out_ref.shape[0]
    intsT = intsT_ref[...]
    featsT = featsT_ref[...]

    rep = jnp.concatenate([intsT] * _NTAB, axis=0)
    val = lax.broadcasted_iota(jnp.int32, (5 * _NTAB, 1), 0) // _NTAB
    ohT = (rep == val).astype(jnp.float32)

    embT = jnp.dot(gmap_ref[...], ohT,
                   preferred_element_type=jnp.float32)
    combT = jnp.concatenate([embT, featsT], axis=0)

    mu = jnp.mean(combT, axis=0, keepdims=True)
    xm = combT - mu
    var = jnp.mean(xm * xm, axis=0, keepdims=True)
    rs = lax.rsqrt(var + _EPS)
    norm2 = jnp.concatenate([xm * rs, jnp.ones((1, R), jnp.float32)],
                            axis=0)

    y = lax.dot_general(norm2.astype(jnp.bfloat16),
                        waug_ref[...].astype(jnp.bfloat16),
                        dimension_numbers=(((0,), (0,)), ((), ())),
                        preferred_element_type=jnp.float32)
    out_ref[...] = jnp.maximum(y, 0.0)


@functools.partial(jax.jit, static_argnames=("block_r",))
def _run(boardInts, boardFeats, twEmb, trEmb, weatherEmb, terrainEmb,
         ln_g, ln_b, W, b, block_r=4096):
    B = boardInts.shape[0]
    intsT = boardInts.T
    featsT = boardFeats.T

    tables = jnp.stack([twEmb, twEmb, trEmb, weatherEmb, terrainEmb])
    t_ckv = jnp.transpose(tables, (0, 2, 1))
    gmap = (t_ckv[:, :, :, None] * jnp.eye(_NTAB, dtype=jnp.float32)[:, None, None, :]
            ).reshape(4 * _NTAB, 5 * _NTAB)

    waug = jnp.concatenate(
        [ln_g[:, None] * W, (ln_b @ W + b)[None, :]], axis=0)

    grid = (B // block_r,)
    full = lambda shape: pl.BlockSpec(shape, lambda i: (0,) * len(shape))
    return pl.pallas_call(
        _board_kernel,
        grid=grid,
        in_specs=[
            pl.BlockSpec((5, block_r), lambda i: (0, i)),
            pl.BlockSpec((_NFEATS, block_r), lambda i: (0, i)),
            full((4 * _NTAB, 5 * _NTAB)),
            full((_NEWDIM + 1, _NHIDDEN)),
        ],
        out_specs=pl.BlockSpec((block_r, _NHIDDEN), lambda i: (i, 0)),
        out_shape=jax.ShapeDtypeStruct((B, _NHIDDEN), jnp.float32),
        compiler_params=pltpu.CompilerParams(
            allow_input_fusion=[True, True, False, False]),
    )(intsT, featsT, gmap, waug)


def kernel(boardInts, boardFeats, twEmb, trEmb, weatherEmb, terrainEmb,
           ln_g, ln_b, W, b):
    return _run(boardInts, boardFeats, twEmb, trEmb, weatherEmb, terrainEmb,
                ln_g, ln_b, W, b)


# speedup vs baseline: 1.0096x; 1.0096x over previous
import functools

import jax
import jax.numpy as jnp
from jax import lax
from jax.experimental import pallas as pl
from jax.experimental.pallas import tpu as pltpu

_NEMB = 4
_NFEATS = 15
_NHIDDEN = 128
_NEWDIM = 3 * _NEMB + _NEMB + _NEMB + _NFEATS
_NTAB = 5
_EPS = 1e-5


def _board_kernel(intsT_ref, featsT_ref, gmap_ref, waug_ref, out_ref):
    R = out_ref.shape[0]
    intsT = intsT_ref[...]
    featsT = featsT_ref[...]

    rep = jnp.concatenate([intsT] * _NTAB, axis=0)
    val = lax.broadcasted_iota(jnp.int32, (5 * _NTAB, 1), 0) // _NTAB
    ohT = (rep == val).astype(jnp.float32)

    embT = jnp.dot(gmap_ref[...], ohT,
                   preferred_element_type=jnp.float32)
    combT = jnp.concatenate([embT, featsT], axis=0)

    mu = jnp.mean(combT, axis=0, keepdims=True)
    xm = combT - mu
    var = jnp.mean(xm * xm, axis=0, keepdims=True)
    rs = lax.rsqrt(var + _EPS)
    norm2 = jnp.concatenate([xm * rs, jnp.ones((1, R), jnp.float32)],
                            axis=0)

    y = lax.dot_general(norm2.astype(jnp.bfloat16),
                        waug_ref[...].astype(jnp.bfloat16),
                        dimension_numbers=(((0,), (0,)), ((), ())),
                        preferred_element_type=jnp.float32)
    out_ref[...] = jnp.maximum(y, 0.0)


@functools.partial(jax.jit, static_argnames=("block_r",))
def _run(boardInts, boardFeats, twEmb, trEmb, weatherEmb, terrainEmb,
         ln_g, ln_b, W, b, block_r=16384):
    B = boardInts.shape[0]
    intsT = boardInts.T
    featsT = boardFeats.T

    tables = jnp.stack([twEmb, twEmb, trEmb, weatherEmb, terrainEmb])
    t_ckv = jnp.transpose(tables, (0, 2, 1))
    gmap = (t_ckv[:, :, :, None] * jnp.eye(_NTAB, dtype=jnp.float32)[:, None, None, :]
            ).reshape(4 * _NTAB, 5 * _NTAB)

    waug = jnp.concatenate(
        [ln_g[:, None] * W, (ln_b @ W + b)[None, :]], axis=0)

    grid = (B // block_r,)
    full = lambda shape: pl.BlockSpec(shape, lambda i: (0,) * len(shape))
    return pl.pallas_call(
        _board_kernel,
        grid=grid,
        in_specs=[
            pl.BlockSpec((5, block_r), lambda i: (0, i)),
            pl.BlockSpec((_NFEATS, block_r), lambda i: (0, i)),
            full((4 * _NTAB, 5 * _NTAB)),
            full((_NEWDIM + 1, _NHIDDEN)),
        ],
        out_specs=pl.BlockSpec((block_r, _NHIDDEN), lambda i: (i, 0)),
        out_shape=jax.ShapeDtypeStruct((B, _NHIDDEN), jnp.float32),
        compiler_params=pltpu.CompilerParams(
            allow_input_fusion=[True, True, False, False]),
    )(intsT, featsT, gmap, waug)


def kernel(boardInts, boardFeats, twEmb, trEmb, weatherEmb, terrainEmb,
           ln_g, ln_b, W, b):
    return _run(boardInts, boardFeats, twEmb, trEmb, weatherEmb, terrainEmb,
                ln_g, ln_b, W, b)


# speedup vs baseline: 1.0958x; 1.0854x over previous
import functools

import jax
import jax.numpy as jnp
from jax import lax
from jax.experimental import pallas as pl
from jax.experimental.pallas import tpu as pltpu

_NEMB = 4
_NFEATS = 15
_NHIDDEN = 128
_NEWDIM = 3 * _NEMB + _NEMB + _NEMB + _NFEATS
_NTAB = 5
_EPS = 1e-5


def _board_kernel(intsT_ref, featsT_ref, gmap_ref, waug_ref, out_ref):
    R = out_ref.shape[0]
    intsT = intsT_ref[...]
    featsT = featsT_ref[...]

    rep = jnp.concatenate([intsT] * _NTAB, axis=0)
    val = lax.broadcasted_iota(jnp.int32, (5 * _NTAB, 1), 0) // _NTAB
    ohT = (rep == val).astype(jnp.float32)

    embT = jnp.dot(gmap_ref[...], ohT,
                   preferred_element_type=jnp.float32)
    combT = jnp.concatenate([embT, featsT], axis=0)

    mu = jnp.mean(combT, axis=0, keepdims=True)
    xm = combT - mu
    var = jnp.mean(xm * xm, axis=0, keepdims=True)
    rs = lax.rsqrt(var + _EPS)
    norm2 = jnp.concatenate([xm * rs, jnp.ones((1, R), jnp.float32)],
                            axis=0)

    y = lax.dot_general(norm2.astype(jnp.bfloat16),
                        waug_ref[...].astype(jnp.bfloat16),
                        dimension_numbers=(((0,), (0,)), ((), ())),
                        preferred_element_type=jnp.float32)
    out_ref[...] = jnp.maximum(y, 0.0)


@functools.partial(jax.jit, static_argnames=("block_r",))
def _run(boardInts, boardFeats, twEmb, trEmb, weatherEmb, terrainEmb,
         ln_g, ln_b, W, b, block_r=8192):
    B = boardInts.shape[0]
    intsT = boardInts.T
    featsT = boardFeats.T

    tables = jnp.stack([twEmb, twEmb, trEmb, weatherEmb, terrainEmb])
    t_ckv = jnp.transpose(tables, (0, 2, 1))
    gmap = (t_ckv[:, :, :, None] * jnp.eye(_NTAB, dtype=jnp.float32)[:, None, None, :]
            ).reshape(4 * _NTAB, 5 * _NTAB)

    waug = jnp.concatenate(
        [ln_g[:, None] * W, (ln_b @ W + b)[None, :]], axis=0)

    grid = (B // block_r,)
    full = lambda shape: pl.BlockSpec(shape, lambda i: (0,) * len(shape))
    return pl.pallas_call(
        _board_kernel,
        grid=grid,
        in_specs=[
            pl.BlockSpec((5, block_r), lambda i: (0, i)),
            pl.BlockSpec((_NFEATS, block_r), lambda i: (0, i)),
            full((4 * _NTAB, 5 * _NTAB)),
            full((_NEWDIM + 1, _NHIDDEN)),
        ],
        out_specs=pl.BlockSpec((block_r, _NHIDDEN), lambda i: (i, 0)),
        out_shape=jax.ShapeDtypeStruct((B, _NHIDDEN), jnp.float32),
        compiler_params=pltpu.CompilerParams(
            allow_input_fusion=[True, True, False, False]),
    )(intsT, featsT, gmap, waug)


def kernel(boardInts, boardFeats, twEmb, trEmb, weatherEmb, terrainEmb,
           ln_g, ln_b, W, b):
    return _run(boardInts, boardFeats, twEmb, trEmb, weatherEmb, terrainEmb,
                ln_g, ln_b, W, b)
